# baseline (speedup 1.0000x reference)
import functools

import jax
import jax.numpy as jnp
import numpy as np
from jax import lax
from jax.experimental import pallas as pl
from jax.experimental.pallas import tpu as pltpu
from jax.experimental.pallas import tpu_sc as plsc

B, C, H, W = 8, 96, 224, 224
N = H * W
BINS = 10
NBIN = N // BINS
NBINP = 5024
RED_CH = NBINP // 16

NTC = 37632
NSC = N - NTC
NB_BLOCKS = 2
NBLOCK = NTC // NB_BLOCKS

NTILES = 4
PER_TILE_B = NTC // NTILES
PER_TILE_A = NSC // NTILES
CH = 112
NCHUNK = PER_TILE_A // CH
GROUPS = CH // 16

_LOG2_COEF = (
    1.477875543e-02, -1.803001920e-01, 9.618684590e-01, -2.945212104e+00,
    5.723410902e+00, -7.443882306e+00, 7.110040005e+00, -3.240703200e+00,
)
_LN2 = 6.93147180559945286e-01


def _tc_body(x_ref, t_ref, nll_ref, bin_ref):
    xb = x_ref[0]
    t = t_ref[0]
    m = jnp.max(xb, axis=0, keepdims=True)
    s = jnp.sum(jnp.exp(xb - m), axis=0, keepdims=True)
    cls = lax.broadcasted_iota(jnp.int32, (C, NBLOCK), 0)
    xt = jnp.sum(jnp.where(cls == t, xb, 0.0), axis=0, keepdims=True)
    logp = xt - m - jnp.log(s)
    nll_ref[0] = -logp
    g = jnp.abs(jnp.exp(logp) - 1.0)
    bf = jnp.floor(g * np.float32(NBIN - 0.0001))
    bin_ref[0] = jnp.minimum(bf, NBIN - 1).astype(jnp.int32)


_tc_stats = pl.pallas_call(
    _tc_body,
    grid=(B * NB_BLOCKS,),
    in_specs=[
        pl.BlockSpec((1, C, NBLOCK), lambda i: (i // NB_BLOCKS, 0, i % NB_BLOCKS)),
        pl.BlockSpec((1, 1, NBLOCK), lambda i: (i // NB_BLOCKS, 0, i % NB_BLOCKS)),
    ],
    out_specs=[
        pl.BlockSpec((1, 1, NBLOCK), lambda i: (i // NB_BLOCKS, 0, i % NB_BLOCKS)),
        pl.BlockSpec((1, 1, NBLOCK), lambda i: (i // NB_BLOCKS, 0, i % NB_BLOCKS)),
    ],
    out_shape=[
        jax.ShapeDtypeStruct((B, 1, NTC), jnp.float32),
        jax.ShapeDtypeStruct((B, 1, NTC), jnp.int32),
    ],
)


def _sc_mesh_params():
    return dict(
        mesh=plsc.VectorSubcoreMesh(core_axis_name="c", subcore_axis_name="s"),
        compiler_params=pltpu.CompilerParams(
            needs_layout_passes=False, use_tc_tiling_on_sc=False
        ),
    )


def _tile_ids():
    cid = lax.axis_index("c")
    sid = lax.axis_index("s")
    b = cid * (16 // NTILES) + sid // NTILES
    member = sid % NTILES
    return sid, b, member


def _zero_repl(repl):
    zero16 = jnp.zeros((16,), jnp.float32)

    def body(j, _):
        for u in range(8):
            repl[pl.ds(j * 128 + u * 16, 16)] = zero16
        return 0

    lax.fori_loop(0, 16 * NBINP // 128, body, 0)


def _reduce_clear(repl, dst, clear):
    zero16 = jnp.zeros((16,), jnp.float32)

    def body(k, _):
        for u in range(2):
            base = k * 32 + u * 16
            acc = repl[pl.ds(base, 16)]
            if clear:
                repl[pl.ds(base, 16)] = zero16
            for l in range(1, 16):
                o = l * NBINP + base
                acc = acc + repl[pl.ds(o, 16)]
                if clear:
                    repl[pl.ds(o, 16)] = zero16
            dst[pl.ds(base, 16)] = acc
        return 0

    lax.fori_loop(0, RED_CH // 2, body, 0)


def _vadd_into(dst, src):
    def body(k, _):
        for u in range(2):
            sl = pl.ds(k * 32 + u * 16, 16)
            dst[sl] = dst[sl] + src[sl]
        return 0

    lax.fori_loop(0, RED_CH // 2, body, 0)


def _spmem_combine(shared, cpart, spart, tmp, sid, member):
    pltpu.sync_copy(cpart, shared.at[sid, 0])
    pltpu.sync_copy(spart, shared.at[sid, 1])
    plsc.subcore_barrier()
    base_slot = sid - member
    for d in range(1, NTILES):
        peer = base_slot + (member + d) % NTILES
        pltpu.sync_copy(shared.at[peer, 0], tmp)
        _vadd_into(cpart, tmp)
        pltpu.sync_copy(shared.at[peer, 1], tmp)
        _vadd_into(spart, tmp)


@functools.cache
def _make_sc_a():
    @functools.partial(
        pl.kernel,
        out_type=jax.ShapeDtypeStruct((B, 2, NBINP), jnp.float32),
        scratch_types=[
            pltpu.VMEM((16 * NBINP,), jnp.float32),
            pltpu.VMEM((C, CH), jnp.float32),
            pltpu.VMEM((CH,), jnp.int32),
            pltpu.VMEM((PER_TILE_A,), jnp.int32),
            pltpu.VMEM((PER_TILE_A,), jnp.float32),
            pltpu.VMEM((NBINP,), jnp.float32),
            pltpu.VMEM((NBINP,), jnp.float32),
            pltpu.VMEM((NBINP,), jnp.float32),
            pltpu.VMEM_SHARED((16, 2, NBINP), jnp.float32),
        ],
        **_sc_mesh_params(),
    )
    def sc_a(x_hbm, tgt_hbm, part_hbm, repl, xbuf, tbuf, binsv, nllv,
             cpart, spart, tmp, shared):
        sid, b, member = _tile_ids()
        p_base = NTC + member * PER_TILE_A

        zero16 = jnp.zeros((16,), jnp.float32)
        ones16 = jnp.full((16,), 1.0, jnp.float32)
        lane_off = lax.broadcasted_iota(jnp.int32, (16,), 0) * NBINP

        _zero_repl(repl)

        def chunk_body(i, _):
            p0 = p_base + i * CH
            pltpu.sync_copy(x_hbm.at[b, :, pl.ds(p0, CH)], xbuf)
            pltpu.sync_copy(tgt_hbm.at[pl.ds(b * N + p0, CH)], tbuf)

            def group_body(g, _):
                pb = g * 16
                tgt = tbuf[pl.ds(pb, 16)]
                m = xbuf[0, pl.ds(pb, 16)]
                for c in range(1, C):
                    m = jnp.maximum(m, xbuf[c, pl.ds(pb, 16)])
                s = zero16
                xt = zero16
                for c in range(C):
                    xv = xbuf[c, pl.ds(pb, 16)]
                    s = s + jnp.exp(xv - m)
                    xt = xt + jnp.where(tgt == c, xv, 0.0)
                bits = plsc.bitcast(s, jnp.int32)
                e = ((bits >> 23) - 127).astype(jnp.float32)
                mant = plsc.bitcast(
                    (bits & 0x007FFFFF) | 0x3F800000, jnp.float32
                )
                p = jnp.full((16,), _LOG2_COEF[0], jnp.float32)
                for coef in _LOG2_COEF[1:]:
                    p = p * mant + np.float32(coef)
                logs = (e + p) * np.float32(_LN2)
                logp = xt - m - logs
                nll = -logp
                gg = jnp.abs(jnp.exp(logp) - 1.0)
                bf = gg * np.float32(NBIN - 0.0001)
                bin16 = jnp.minimum(bf.astype(jnp.int32), NBIN - 1)
                off = i * CH + pb
                binsv[pl.ds(off, 16)] = bin16
                nllv[pl.ds(off, 16)] = nll
                plsc.addupdate_scatter(repl, [lane_off + bin16], ones16)
                return 0

            lax.fori_loop(0, GROUPS, group_body, 0)
            return 0

        lax.fori_loop(0, NCHUNK, chunk_body, 0)

        _reduce_clear(repl, cpart, clear=True)

        def scat_s(i, _):
            for u in range(2):
                sl = pl.ds(i * 32 + u * 16, 16)
                plsc.addupdate_scatter(repl, [lane_off + binsv[sl]], nllv[sl])
            return 0

        lax.fori_loop(0, PER_TILE_A // 32, scat_s, 0)
        _reduce_clear(repl, spart, clear=False)

        _spmem_combine(shared, cpart, spart, tmp, sid, member)

        @pl.when(member == 0)
        def _():
            pltpu.sync_copy(cpart, part_hbm.at[b, 0])
            pltpu.sync_copy(spart, part_hbm.at[b, 1])

    return sc_a


@functools.cache
def _make_sc_b():
    @functools.partial(
        pl.kernel,
        out_type=jax.ShapeDtypeStruct((B, 16), jnp.float32),
        scratch_types=[
            pltpu.VMEM((16 * NBINP,), jnp.float32),
            pltpu.VMEM((PER_TILE_B,), jnp.int32),
            pltpu.VMEM((PER_TILE_B,), jnp.float32),
            pltpu.VMEM((NBINP,), jnp.float32),
            pltpu.VMEM((NBINP,), jnp.float32),
            pltpu.VMEM((NBINP,), jnp.float32),
            pltpu.VMEM((16,), jnp.float32),
            pltpu.VMEM_SHARED((16, 2, NBINP), jnp.float32),
        ],
        **_sc_mesh_params(),
    )
    def sc_b(bins_hbm, nll_hbm, part_hbm, out_hbm, repl, binsv, nllv,
             cpart, spart, tmp, outv, shared):
        sid, b, member = _tile_ids()
        off = b * NTC + member * PER_TILE_B

        zero16 = jnp.zeros((16,), jnp.float32)
        ones16 = jnp.full((16,), 1.0, jnp.float32)
        lane_off = lax.broadcasted_iota(jnp.int32, (16,), 0) * NBINP

        _zero_repl(repl)

        pltpu.sync_copy(bins_hbm.at[pl.ds(off, PER_TILE_B)], binsv)
        pltpu.sync_copy(nll_hbm.at[pl.ds(off, PER_TILE_B)], nllv)

        def scat_c(i, _):
            for u in range(2):
                idx = binsv[pl.ds(i * 32 + u * 16, 16)]
                plsc.addupdate_scatter(repl, [lane_off + idx], ones16)
            return 0

        lax.fori_loop(0, PER_TILE_B // 32, scat_c, 0)
        _reduce_clear(repl, cpart, clear=True)

        def scat_s(i, _):
            for u in range(2):
                sl = pl.ds(i * 32 + u * 16, 16)
                plsc.addupdate_scatter(repl, [lane_off + binsv[sl]], nllv[sl])
            return 0

        lax.fori_loop(0, PER_TILE_B // 32, scat_s, 0)
        _reduce_clear(repl, spart, clear=False)

        _spmem_combine(shared, cpart, spart, tmp, sid, member)

        pltpu.sync_copy(part_hbm.at[b, 0], tmp)
        _vadd_into(cpart, tmp)
        pltpu.sync_copy(part_hbm.at[b, 1], tmp)
        _vadd_into(spart, tmp)

        @pl.when(member == 0)
        def _():
            def fin(k, carry):
                ne_a, t_a = carry
                for u in range(2):
                    sl = pl.ds(k * 32 + u * 16, 16)
                    cc = cpart[sl]
                    ss = spart[sl]
                    ne_a = ne_a + jnp.where(cc > 0.0, 1.0, 0.0)
                    t_a = t_a + ss / jnp.maximum(cc, 1.0)
                return ne_a, t_a

            ne16, term16 = lax.fori_loop(0, RED_CH // 2, fin, (zero16, zero16))
            term_v = zero16 + jnp.sum(term16)
            ne_v = zero16 + jnp.sum(ne16)
            outv[...] = term_v / ne_v
            pltpu.sync_copy(outv, out_hbm.at[b])

    return sc_b


def kernel(x, target):
    x3 = x.reshape(B, C, N)
    t3 = target.reshape(B, 1, N)
    part = _make_sc_a()(x3, target.reshape(-1))
    nll3, bin3 = _tc_stats(x3, t3)
    per_sample = _make_sc_b()(bin3.reshape(-1), nll3.reshape(-1), part)
    return jnp.mean(per_sample[:, 0])


# speedup vs baseline: 1.6899x; 1.6899x over previous
import functools

import jax
import jax.numpy as jnp
import numpy as np
from jax import lax
from jax.experimental import pallas as pl
from jax.experimental.pallas import tpu as pltpu
from jax.experimental.pallas import tpu_sc as plsc

B, C, H, W = 8, 96, 224, 224
N = H * W
BINS = 10
NBIN = N // BINS
NBINP = 5024
NB_BLOCKS = 2
NBLOCK = N // NB_BLOCKS
NTILES = 4
PER_TILE = N // NTILES
RED_CH = NBINP // 16


def _tc_body(x_ref, t_ref, nll_ref, bin_ref):
    xb = x_ref[0]
    t = t_ref[0]
    m = jnp.max(xb, axis=0, keepdims=True)
    s = jnp.sum(jnp.exp(xb - m), axis=0, keepdims=True)
    cls = lax.broadcasted_iota(jnp.int32, (C, NBLOCK), 0)
    xt = jnp.sum(jnp.where(cls == t, xb, 0.0), axis=0, keepdims=True)
    logp = xt - m - jnp.log(s)
    nll_ref[0] = -logp
    g = jnp.abs(jnp.exp(logp) - 1.0)
    bf = jnp.floor(g * np.float32(NBIN - 0.0001))
    bin_ref[0] = jnp.minimum(bf, NBIN - 1).astype(jnp.int32)


_tc_stats = pl.pallas_call(
    _tc_body,
    grid=(B * NB_BLOCKS,),
    in_specs=[
        pl.BlockSpec((1, C, NBLOCK), lambda i: (i // NB_BLOCKS, 0, i % NB_BLOCKS)),
        pl.BlockSpec((1, 1, NBLOCK), lambda i: (i // NB_BLOCKS, 0, i % NB_BLOCKS)),
    ],
    out_specs=[
        pl.BlockSpec((1, 1, NBLOCK), lambda i: (i // NB_BLOCKS, 0, i % NB_BLOCKS)),
        pl.BlockSpec((1, 1, NBLOCK), lambda i: (i // NB_BLOCKS, 0, i % NB_BLOCKS)),
    ],
    out_shape=[
        jax.ShapeDtypeStruct((B, 1, N), jnp.float32),
        jax.ShapeDtypeStruct((B, 1, N), jnp.int32),
    ],
)


@functools.cache
def _make_sc_kernel():
    mesh = plsc.VectorSubcoreMesh(core_axis_name="c", subcore_axis_name="s")

    @functools.partial(
        pl.kernel,
        out_type=jax.ShapeDtypeStruct((B, 16), jnp.float32),
        mesh=mesh,
        compiler_params=pltpu.CompilerParams(
            needs_layout_passes=False, use_tc_tiling_on_sc=False
        ),
        scratch_types=[
            pltpu.VMEM((16 * NBINP,), jnp.float32),
            pltpu.VMEM((PER_TILE,), jnp.int32),
            pltpu.VMEM((PER_TILE,), jnp.float32),
            pltpu.VMEM((NBINP,), jnp.float32),
            pltpu.VMEM((NBINP,), jnp.float32),
            pltpu.VMEM((NBINP,), jnp.float32),
            pltpu.VMEM((16,), jnp.float32),
            pltpu.VMEM_SHARED((16, 2, NBINP), jnp.float32),
        ],
    )
    def sc_hist(bins_hbm, nll_hbm, out_hbm, repl, binsv, nllv, cpart, spart,
                tmp, outv, shared):
        cid = lax.axis_index("c")
        sid = lax.axis_index("s")
        b = cid * (16 // NTILES) + sid // NTILES
        member = sid % NTILES
        off = b * N + member * PER_TILE

        zero16 = jnp.zeros((16,), jnp.float32)
        ones16 = jnp.full((16,), 1.0, jnp.float32)
        lane = lax.broadcasted_iota(jnp.int32, (16,), 0)

        def zero_body(j, _):
            for u in range(8):
                repl[pl.ds(j * 128 + u * 16, 16)] = zero16
            return 0

        lax.fori_loop(0, 16 * NBINP // 128, zero_body, 0)

        pltpu.sync_copy(bins_hbm.at[pl.ds(off, PER_TILE)], binsv)
        pltpu.sync_copy(nll_hbm.at[pl.ds(off, PER_TILE)], nllv)

        lane_off = lane * NBINP

        def scat_c(i, _):
            for u in range(2):
                idx = binsv[pl.ds(i * 32 + u * 16, 16)]
                plsc.addupdate_scatter(repl, [lane_off + idx], ones16)
            return 0

        lax.fori_loop(0, PER_TILE // 32, scat_c, 0)

        def red_c(k, _):
            for u in range(2):
                base = k * 32 + u * 16
                acc = repl[pl.ds(base, 16)]
                repl[pl.ds(base, 16)] = zero16
                for l in range(1, 16):
                    o = l * NBINP + base
                    acc = acc + repl[pl.ds(o, 16)]
                    repl[pl.ds(o, 16)] = zero16
                cpart[pl.ds(base, 16)] = acc
            return 0

        lax.fori_loop(0, RED_CH // 2, red_c, 0)

        def scat_s(i, _):
            for u in range(2):
                idx = binsv[pl.ds(i * 32 + u * 16, 16)]
                vals = nllv[pl.ds(i * 32 + u * 16, 16)]
                plsc.addupdate_scatter(repl, [lane_off + idx], vals)
            return 0

        lax.fori_loop(0, PER_TILE // 32, scat_s, 0)

        def red_s(k, _):
            for u in range(2):
                base = k * 32 + u * 16
                acc = repl[pl.ds(base, 16)]
                for l in range(1, 16):
                    acc = acc + repl[pl.ds(l * NBINP + base, 16)]
                spart[pl.ds(base, 16)] = acc
            return 0

        lax.fori_loop(0, RED_CH // 2, red_s, 0)

        pltpu.sync_copy(cpart, shared.at[sid, 0])
        pltpu.sync_copy(spart, shared.at[sid, 1])
        plsc.subcore_barrier()

        base_slot = sid - member
        for d in range(1, NTILES):
            peer = base_slot + (member + d) % NTILES
            pltpu.sync_copy(shared.at[peer, 0], tmp)

            def addc(k, _):
                for u in range(2):
                    sl = pl.ds(k * 32 + u * 16, 16)
                    cpart[sl] = cpart[sl] + tmp[sl]
                return 0

            lax.fori_loop(0, RED_CH // 2, addc, 0)
            pltpu.sync_copy(shared.at[peer, 1], tmp)

            def adds(k, _):
                for u in range(2):
                    sl = pl.ds(k * 32 + u * 16, 16)
                    spart[sl] = spart[sl] + tmp[sl]
                return 0

            lax.fori_loop(0, RED_CH // 2, adds, 0)

        @pl.when(member == 0)
        def _():
            def fin(k, carry):
                ne_a, t_a = carry
                for u in range(2):
                    sl = pl.ds(k * 32 + u * 16, 16)
                    cc = cpart[sl]
                    ss = spart[sl]
                    ne_a = ne_a + jnp.where(cc > 0.0, 1.0, 0.0)
                    t_a = t_a + ss / jnp.maximum(cc, 1.0)
                return ne_a, t_a

            ne16, term16 = lax.fori_loop(0, RED_CH // 2, fin, (zero16, zero16))
            term_v = zero16 + jnp.sum(term16)
            ne_v = zero16 + jnp.sum(ne16)
            outv[...] = term_v / ne_v
            pltpu.sync_copy(outv, out_hbm.at[b])

    return sc_hist


def kernel(x, target):
    x3 = x.reshape(B, C, N)
    t3 = target.reshape(B, 1, N)
    nll3, bin3 = _tc_stats(x3, t3)
    per_sample = _make_sc_kernel()(bin3.reshape(-1), nll3.reshape(-1))
    return jnp.mean(per_sample[:, 0])
